# initial kernel scaffold (unmeasured)
import jax
import jax.numpy as jnp
from jax import lax
from jax.experimental import pallas as pl
from jax.experimental.pallas import tpu as pltpu


def kernel(
    x,
):
    def body(*refs):
        pass

    out_shape = jax.ShapeDtypeStruct(..., jnp.float32)
    return pl.pallas_call(body, out_shape=out_shape)(...)


# baseline (device time: 46126 ns/iter reference)
import jax
import jax.numpy as jnp
from jax import lax
from jax.experimental import pallas as pl
from jax.experimental.pallas import tpu as pltpu

N_DEV = 4
M = 512
N_TOTAL = 2048
CHUNK = 512


def kernel(x):
    x = x.reshape(M, N_TOTAL)

    def body(x_ref, out_ref, comm_ref, send_sems, recv_sems):
        my_x = lax.axis_index("x")
        my_y = lax.axis_index("y")
        p = lax.axis_index("z")
        right = (p + 1) % N_DEV
        left = (p + 3) % N_DEV

        barrier_sem = pltpu.get_barrier_semaphore()
        for nbr in (left, right):
            pl.semaphore_signal(
                barrier_sem, inc=1,
                device_id=(my_x, my_y, nbr),
                device_id_type=pl.DeviceIdType.MESH,
            )
        pl.semaphore_wait(barrier_sem, 2)

        idx0 = (p + 3) % N_DEV
        comm_ref[0, :, :] = x_ref[:, pl.ds(idx0 * CHUNK, CHUNK)]

        for h in range(N_DEV - 1):
            rdma = pltpu.make_async_remote_copy(
                src_ref=comm_ref.at[h],
                dst_ref=comm_ref.at[h + 1],
                send_sem=send_sems.at[h],
                recv_sem=recv_sems.at[h],
                device_id=(my_x, my_y, right),
                device_id_type=pl.DeviceIdType.MESH,
            )
            rdma.start()
            rdma.wait()

            idx = (p + 2 - h) % N_DEV
            local = x_ref[:, pl.ds(idx * CHUNK, CHUNK)]
            if h < N_DEV - 2:
                comm_ref[h + 1, :, :] = comm_ref[h + 1, :, :] + local
            else:
                out_ref[:, :] = comm_ref[h + 1, :, :] + local

    return pl.pallas_call(
        body,
        out_shape=jax.ShapeDtypeStruct((M, CHUNK), jnp.float32),
        in_specs=[pl.BlockSpec(memory_space=pltpu.VMEM)],
        out_specs=pl.BlockSpec(memory_space=pltpu.VMEM),
        scratch_shapes=[
            pltpu.VMEM((N_DEV, M, CHUNK), jnp.float32),
            pltpu.SemaphoreType.DMA((N_DEV - 1,)),
            pltpu.SemaphoreType.DMA((N_DEV - 1,)),
        ],
        compiler_params=pltpu.CompilerParams(collective_id=0),
    )(x)


# device time: 31593 ns/iter; 1.4600x vs baseline; 1.4600x over previous
import functools

import jax
import jax.numpy as jnp
from jax import lax
from jax.experimental import pallas as pl
from jax.experimental.pallas import tpu as pltpu

N_Z = 4
M = 512
N_TOTAL = 2048
CHUNK = 512
QROWS = M // 4


def kernel(x):
    x = x.reshape(4, QROWS, N_TOTAL)

    def body(x_ref, out_ref, comm_ref, xybuf_ref,
             z_send_sems, z_recv_sems, xy_send_sems, xy_recv_sems):
        mx = lax.axis_index("x")
        my = lax.axis_index("y")
        p = lax.axis_index("z")
        ox = 1 - mx
        oy = 1 - my
        right = (p + 1) % N_Z
        left = (p + 3) % N_Z
        q = 2 * mx + my

        peers = [
            (mx, my, left),
            (mx, my, right),
            (ox, my, p),
            (mx, oy, p),
            (ox, oy, p),
        ]

        barrier_sem = pltpu.get_barrier_semaphore()
        for dev in peers:
            pl.semaphore_signal(
                barrier_sem, inc=1,
                device_id=dev, device_id_type=pl.DeviceIdType.MESH,
            )
        pl.semaphore_wait(barrier_sem, len(peers))

        idx0 = (p + 3) % N_Z
        comm_ref[0, :, :] = x_ref[q, :, pl.ds(idx0 * CHUNK, CHUNK)]

        for h in range(N_Z - 1):
            rdma = pltpu.make_async_remote_copy(
                src_ref=comm_ref.at[h],
                dst_ref=comm_ref.at[h + 1],
                send_sem=z_send_sems.at[h],
                recv_sem=z_recv_sems.at[h],
                device_id=(mx, my, right),
                device_id_type=pl.DeviceIdType.MESH,
            )
            rdma.start()
            rdma.wait()

            idx = (p + 2 - h) % N_Z
            local = x_ref[q, :, pl.ds(idx * CHUNK, CHUNK)]
            if h < N_Z - 2:
                comm_ref[h + 1, :, :] = comm_ref[h + 1, :, :] + local
            else:
                xybuf_ref[:, :] = comm_ref[h + 1, :, :] + local

        xy_peers = [(ox, my, p), (mx, oy, p), (ox, oy, p)]
        sends = []
        for k, dev in enumerate(xy_peers):
            send = pltpu.make_async_remote_copy(
                src_ref=xybuf_ref,
                dst_ref=out_ref.at[q],
                send_sem=xy_send_sems.at[k],
                recv_sem=xy_recv_sems.at[k],
                device_id=dev,
                device_id_type=pl.DeviceIdType.MESH,
            )
            send.start()
            sends.append(send)

        out_ref[q, :, :] = xybuf_ref[:, :]

        peer_qids = [2 * ox + my, 2 * mx + oy, 2 * ox + oy]
        for k, qk in enumerate(peer_qids):
            recv = pltpu.make_async_remote_copy(
                src_ref=xybuf_ref,
                dst_ref=out_ref.at[qk],
                send_sem=xy_send_sems.at[k],
                recv_sem=xy_recv_sems.at[k],
                device_id=xy_peers[k],
                device_id_type=pl.DeviceIdType.MESH,
            )
            recv.wait_recv()
        for send in sends:
            send.wait_send()

        @functools.partial(pl.run_scoped, exit_sem=pltpu.SemaphoreType.REGULAR)
        def _(exit_sem):
            for dev in peers:
                pl.semaphore_signal(
                    exit_sem, inc=1,
                    device_id=dev, device_id_type=pl.DeviceIdType.MESH,
                )
            pl.semaphore_wait(exit_sem, len(peers))

    out = pl.pallas_call(
        body,
        out_shape=jax.ShapeDtypeStruct((4, QROWS, CHUNK), jnp.float32),
        in_specs=[pl.BlockSpec(memory_space=pltpu.VMEM)],
        out_specs=pl.BlockSpec(memory_space=pltpu.VMEM),
        scratch_shapes=[
            pltpu.VMEM((N_Z, QROWS, CHUNK), jnp.float32),
            pltpu.VMEM((QROWS, CHUNK), jnp.float32),
            pltpu.SemaphoreType.DMA((N_Z - 1,)),
            pltpu.SemaphoreType.DMA((N_Z - 1,)),
            pltpu.SemaphoreType.DMA((3,)),
            pltpu.SemaphoreType.DMA((3,)),
        ],
        compiler_params=pltpu.CompilerParams(collective_id=0),
    )(x)
    return out.reshape(M, CHUNK)


# device time: 26775 ns/iter; 1.7227x vs baseline; 1.1799x over previous
import functools

import jax
import jax.numpy as jnp
from jax import lax
from jax.experimental import pallas as pl
from jax.experimental.pallas import tpu as pltpu

N_Z = 4
M = 512
N_TOTAL = 2048
CHUNK = 512
QROWS = M // 4
S = 2
SUBR = QROWS // S


def _gray(v):
    return jnp.bitwise_xor(v, jnp.right_shift(v, 1))


def kernel(x):
    x = x.reshape(4, S, SUBR, N_TOTAL)

    def body(x_ref, out_ref, comm_ref, xybuf_ref,
             z_send_sems, z_recv_sems, xy_send_sems, xy_recv_sems):
        mx = lax.axis_index("x")
        my = lax.axis_index("y")
        p = lax.axis_index("z")
        ox = 1 - mx
        oy = 1 - my
        q = 2 * mx + my

        r = _gray(p)
        succ = _gray((r + 1) % N_Z)
        pred = _gray((r + 3) % N_Z)

        peers = [
            (mx, my, pred),
            (mx, my, succ),
            (ox, my, p),
            (mx, oy, p),
            (ox, oy, p),
        ]

        barrier_sem = pltpu.get_barrier_semaphore()
        for dev in peers:
            pl.semaphore_signal(
                barrier_sem, inc=1,
                device_id=dev, device_id_type=pl.DeviceIdType.MESH,
            )
        pl.semaphore_wait(barrier_sem, len(peers))

        def local_chunk(h, s):
            c = _gray((r + 2 - h) % N_Z)
            return x_ref[q, s, :, pl.ds(c * CHUNK, CHUNK)]

        c0 = _gray((r + 3) % N_Z)
        for s in range(S):
            comm_ref[0, s, :, :] = x_ref[q, s, :, pl.ds(c0 * CHUNK, CHUNK)]

        def z_rdma(h, s):
            return pltpu.make_async_remote_copy(
                src_ref=comm_ref.at[h, s],
                dst_ref=comm_ref.at[h + 1, s],
                send_sem=z_send_sems.at[h, s],
                recv_sem=z_recv_sems.at[h, s],
                device_id=(mx, my, succ),
                device_id_type=pl.DeviceIdType.MESH,
            )

        xy_peers = [(ox, my, p), (mx, oy, p), (ox, oy, p)]
        peer_qids = [2 * ox + my, 2 * mx + oy, 2 * ox + oy]
        z_sends = []
        xy_sends = []

        for s in range(S):
            rd = z_rdma(0, s)
            rd.start()
            z_sends.append(rd)

        for h in range(N_Z - 1):
            for s in range(S):
                z_sends[h * S + s].wait_recv()
                if h < N_Z - 2:
                    comm_ref[h + 1, s, :, :] = (
                        comm_ref[h + 1, s, :, :] + local_chunk(h, s)
                    )
                    rd = z_rdma(h + 1, s)
                    rd.start()
                    z_sends.append(rd)
                else:
                    xybuf_ref[s, :, :] = comm_ref[h + 1, s, :, :] + local_chunk(h, s)
                    out_ref[q, s, :, :] = xybuf_ref[s, :, :]
                    for k, dev in enumerate(xy_peers):
                        send = pltpu.make_async_remote_copy(
                            src_ref=xybuf_ref.at[s],
                            dst_ref=out_ref.at[q, s],
                            send_sem=xy_send_sems.at[k, s],
                            recv_sem=xy_recv_sems.at[k, s],
                            device_id=dev,
                            device_id_type=pl.DeviceIdType.MESH,
                        )
                        send.start()
                        xy_sends.append(send)

        for k in range(3):
            for s in range(S):
                recv = pltpu.make_async_remote_copy(
                    src_ref=xybuf_ref.at[s],
                    dst_ref=out_ref.at[peer_qids[k], s],
                    send_sem=xy_send_sems.at[k, s],
                    recv_sem=xy_recv_sems.at[k, s],
                    device_id=xy_peers[k],
                    device_id_type=pl.DeviceIdType.MESH,
                )
                recv.wait_recv()

        for rd in z_sends:
            rd.wait_send()
        for rd in xy_sends:
            rd.wait_send()

        @functools.partial(pl.run_scoped, exit_sem=pltpu.SemaphoreType.REGULAR)
        def _(exit_sem):
            for dev in peers:
                pl.semaphore_signal(
                    exit_sem, inc=1,
                    device_id=dev, device_id_type=pl.DeviceIdType.MESH,
                )
            pl.semaphore_wait(exit_sem, len(peers))

    out = pl.pallas_call(
        body,
        out_shape=jax.ShapeDtypeStruct((4, S, SUBR, CHUNK), jnp.float32),
        in_specs=[pl.BlockSpec(memory_space=pltpu.VMEM)],
        out_specs=pl.BlockSpec(memory_space=pltpu.VMEM),
        scratch_shapes=[
            pltpu.VMEM((N_Z, S, SUBR, CHUNK), jnp.float32),
            pltpu.VMEM((S, SUBR, CHUNK), jnp.float32),
            pltpu.SemaphoreType.DMA((N_Z - 1, S)),
            pltpu.SemaphoreType.DMA((N_Z - 1, S)),
            pltpu.SemaphoreType.DMA((3, S)),
            pltpu.SemaphoreType.DMA((3, S)),
        ],
        compiler_params=pltpu.CompilerParams(collective_id=0),
    )(x)
    return out.reshape(M, CHUNK)


# device time: 25706 ns/iter; 1.7944x vs baseline; 1.0416x over previous
import functools

import jax
import jax.numpy as jnp
from jax import lax
from jax.experimental import pallas as pl
from jax.experimental.pallas import tpu as pltpu

N_Z = 4
M = 512
N_TOTAL = 2048
CHUNK = 512
QROWS = M // 4
S = 4
SUBR = QROWS // S


def _gray(v):
    return jnp.bitwise_xor(v, jnp.right_shift(v, 1))


def kernel(x):
    x = x.reshape(4, S, SUBR, N_TOTAL)

    def body(x_ref, out_ref, xq_ref, comm_ref, xybuf_ref, copy_sem,
             z_send_sems, z_recv_sems, xy_send_sems, xy_recv_sems):
        mx = lax.axis_index("x")
        my = lax.axis_index("y")
        p = lax.axis_index("z")
        ox = 1 - mx
        oy = 1 - my
        q = 2 * mx + my

        r = _gray(p)
        succ = _gray((r + 1) % N_Z)
        pred = _gray((r + 3) % N_Z)

        stage = pltpu.make_async_copy(x_ref.at[q], xq_ref, copy_sem)
        stage.start()

        peers = [
            (mx, my, pred),
            (mx, my, succ),
            (ox, my, p),
            (mx, oy, p),
            (ox, oy, p),
        ]

        barrier_sem = pltpu.get_barrier_semaphore()
        for dev in peers:
            pl.semaphore_signal(
                barrier_sem, inc=1,
                device_id=dev, device_id_type=pl.DeviceIdType.MESH,
            )
        pl.semaphore_wait(barrier_sem, len(peers))
        stage.wait()

        def local_chunk(h, s):
            c = _gray((r + 2 - h) % N_Z)
            return xq_ref[s, :, pl.ds(c * CHUNK, CHUNK)]

        def z_rdma(h, s):
            return pltpu.make_async_remote_copy(
                src_ref=comm_ref.at[h, s],
                dst_ref=comm_ref.at[h + 1, s],
                send_sem=z_send_sems.at[h, s],
                recv_sem=z_recv_sems.at[h, s],
                device_id=(mx, my, succ),
                device_id_type=pl.DeviceIdType.MESH,
            )

        xy_peers = [(ox, my, p), (mx, oy, p), (ox, oy, p)]
        peer_qids = [2 * ox + my, 2 * mx + oy, 2 * ox + oy]
        z_sends = []
        xy_sends = []

        c0 = _gray((r + 3) % N_Z)
        for s in range(S):
            comm_ref[0, s, :, :] = xq_ref[s, :, pl.ds(c0 * CHUNK, CHUNK)]
            rd = z_rdma(0, s)
            rd.start()
            z_sends.append(rd)

        for h in range(N_Z - 1):
            for s in range(S):
                z_sends[h * S + s].wait_recv()
                if h < N_Z - 2:
                    comm_ref[h + 1, s, :, :] = (
                        comm_ref[h + 1, s, :, :] + local_chunk(h, s)
                    )
                    rd = z_rdma(h + 1, s)
                    rd.start()
                    z_sends.append(rd)
                else:
                    xybuf_ref[s, :, :] = comm_ref[h + 1, s, :, :] + local_chunk(h, s)
                    out_ref[q, s, :, :] = xybuf_ref[s, :, :]
                    for k, dev in enumerate(xy_peers):
                        send = pltpu.make_async_remote_copy(
                            src_ref=xybuf_ref.at[s],
                            dst_ref=out_ref.at[q, s],
                            send_sem=xy_send_sems.at[k, s],
                            recv_sem=xy_recv_sems.at[k, s],
                            device_id=dev,
                            device_id_type=pl.DeviceIdType.MESH,
                        )
                        send.start()
                        xy_sends.append(send)

        for k in range(3):
            for s in range(S):
                recv = pltpu.make_async_remote_copy(
                    src_ref=xybuf_ref.at[s],
                    dst_ref=out_ref.at[peer_qids[k], s],
                    send_sem=xy_send_sems.at[k, s],
                    recv_sem=xy_recv_sems.at[k, s],
                    device_id=xy_peers[k],
                    device_id_type=pl.DeviceIdType.MESH,
                )
                recv.wait_recv()

        for rd in z_sends:
            rd.wait_send()
        for rd in xy_sends:
            rd.wait_send()

        @functools.partial(pl.run_scoped, exit_sem=pltpu.SemaphoreType.REGULAR)
        def _(exit_sem):
            for dev in peers:
                pl.semaphore_signal(
                    exit_sem, inc=1,
                    device_id=dev, device_id_type=pl.DeviceIdType.MESH,
                )
            pl.semaphore_wait(exit_sem, len(peers))

    out = pl.pallas_call(
        body,
        out_shape=jax.ShapeDtypeStruct((4, S, SUBR, CHUNK), jnp.float32),
        in_specs=[pl.BlockSpec(memory_space=pl.ANY)],
        out_specs=pl.BlockSpec(memory_space=pltpu.VMEM),
        scratch_shapes=[
            pltpu.VMEM((S, SUBR, N_TOTAL), jnp.float32),
            pltpu.VMEM((N_Z, S, SUBR, CHUNK), jnp.float32),
            pltpu.VMEM((S, SUBR, CHUNK), jnp.float32),
            pltpu.SemaphoreType.DMA,
            pltpu.SemaphoreType.DMA((N_Z - 1, S)),
            pltpu.SemaphoreType.DMA((N_Z - 1, S)),
            pltpu.SemaphoreType.DMA((3, S)),
            pltpu.SemaphoreType.DMA((3, S)),
        ],
        compiler_params=pltpu.CompilerParams(collective_id=0),
    )(x)
    return out.reshape(M, CHUNK)
